# baseline (device time: 20264 ns/iter reference)
import jax
import jax.numpy as jnp
from jax import lax
from jax.experimental import pallas as pl
from jax.experimental.pallas import tpu as pltpu

N_Z = 2


def kernel(x):
    _, m, n_total = x.shape
    half = n_total // N_Z
    rows = m // 2

    def body(x_ref, out_ref, recv_z_ref, sem_sz, sem_rz, sem_sx, sem_rx):
        my_x = lax.axis_index("x")
        my_y = lax.axis_index("y")
        my_z = lax.axis_index("z")
        other_z = 1 - my_z
        other_x = 1 - my_x
        r0 = my_x * rows

        barrier_sem = pltpu.get_barrier_semaphore()
        pl.semaphore_signal(
            barrier_sem, inc=1,
            device_id=(my_x, my_y, other_z),
            device_id_type=pl.DeviceIdType.MESH,
        )
        pl.semaphore_signal(
            barrier_sem, inc=1,
            device_id=(other_x, my_y, my_z),
            device_id_type=pl.DeviceIdType.MESH,
        )
        pl.semaphore_wait(barrier_sem, 2)

        rdma_z = pltpu.make_async_remote_copy(
            src_ref=x_ref.at[0, pl.ds(r0, rows), pl.ds(other_z * half, half)],
            dst_ref=recv_z_ref,
            send_sem=sem_sz,
            recv_sem=sem_rz,
            device_id=(my_x, my_y, other_z),
            device_id_type=pl.DeviceIdType.MESH,
        )
        rdma_z.start()
        rdma_z.wait()

        out_ref[pl.ds(r0, rows), :] = (
            x_ref[0, pl.ds(r0, rows), pl.ds(my_z * half, half)]
            + recv_z_ref[:, :]
        )

        rdma_x = pltpu.make_async_remote_copy(
            src_ref=out_ref.at[pl.ds(r0, rows), :],
            dst_ref=out_ref.at[pl.ds(r0, rows), :],
            send_sem=sem_sx,
            recv_sem=sem_rx,
            device_id=(other_x, my_y, my_z),
            device_id_type=pl.DeviceIdType.MESH,
        )
        rdma_x.start()
        rdma_x.wait()

    return pl.pallas_call(
        body,
        out_shape=jax.ShapeDtypeStruct((m, half), jnp.float32),
        in_specs=[pl.BlockSpec(memory_space=pltpu.VMEM)],
        out_specs=pl.BlockSpec(memory_space=pltpu.VMEM),
        scratch_shapes=[
            pltpu.VMEM((rows, half), jnp.float32),
            pltpu.SemaphoreType.DMA,
            pltpu.SemaphoreType.DMA,
            pltpu.SemaphoreType.DMA,
            pltpu.SemaphoreType.DMA,
        ],
        compiler_params=pltpu.CompilerParams(collective_id=0),
    )(x)


# device time: 16078 ns/iter; 1.2604x vs baseline; 1.2604x over previous
import jax
import jax.numpy as jnp
from jax import lax
from jax.experimental import pallas as pl
from jax.experimental.pallas import tpu as pltpu

N_Z = 2
C = 4


def kernel(x):
    _, m, n_total = x.shape
    half = n_total // N_Z
    rows = m // 2
    cr = rows // C

    def body(x_ref, out_ref, recv_z_ref, sem_sz, sem_rz, sem_sx, sem_rx):
        my_x = lax.axis_index("x")
        my_y = lax.axis_index("y")
        my_z = lax.axis_index("z")
        other_z = 1 - my_z
        other_x = 1 - my_x
        r0 = my_x * rows

        barrier_sem = pltpu.get_barrier_semaphore()
        pl.semaphore_signal(
            barrier_sem, inc=1,
            device_id=(my_x, my_y, other_z),
            device_id_type=pl.DeviceIdType.MESH,
        )
        pl.semaphore_signal(
            barrier_sem, inc=1,
            device_id=(other_x, my_y, my_z),
            device_id_type=pl.DeviceIdType.MESH,
        )
        pl.semaphore_wait(barrier_sem, 2)

        rdma_z = []
        for i in range(C):
            d = pltpu.make_async_remote_copy(
                src_ref=x_ref.at[
                    0, pl.ds(r0 + i * cr, cr), pl.ds(other_z * half, half)
                ],
                dst_ref=recv_z_ref.at[i],
                send_sem=sem_sz.at[i],
                recv_sem=sem_rz.at[i],
                device_id=(my_x, my_y, other_z),
                device_id_type=pl.DeviceIdType.MESH,
            )
            d.start()
            rdma_z.append(d)

        rdma_x = []
        for i in range(C):
            rdma_z[i].wait_recv()
            out_ref[pl.ds(r0 + i * cr, cr), :] = (
                x_ref[0, pl.ds(r0 + i * cr, cr), pl.ds(my_z * half, half)]
                + recv_z_ref[i, :, :]
            )
            d = pltpu.make_async_remote_copy(
                src_ref=out_ref.at[pl.ds(r0 + i * cr, cr), :],
                dst_ref=out_ref.at[pl.ds(r0 + i * cr, cr), :],
                send_sem=sem_sx.at[i],
                recv_sem=sem_rx.at[i],
                device_id=(other_x, my_y, my_z),
                device_id_type=pl.DeviceIdType.MESH,
            )
            d.start()
            rdma_x.append(d)

        for i in range(C):
            rdma_z[i].wait_send()
            rdma_x[i].wait()

    return pl.pallas_call(
        body,
        out_shape=jax.ShapeDtypeStruct((m, half), jnp.float32),
        in_specs=[pl.BlockSpec(memory_space=pltpu.VMEM)],
        out_specs=pl.BlockSpec(memory_space=pltpu.VMEM),
        scratch_shapes=[
            pltpu.VMEM((C, cr, half), jnp.float32),
            pltpu.SemaphoreType.DMA((C,)),
            pltpu.SemaphoreType.DMA((C,)),
            pltpu.SemaphoreType.DMA((C,)),
            pltpu.SemaphoreType.DMA((C,)),
        ],
        compiler_params=pltpu.CompilerParams(collective_id=0),
    )(x)
